# baseline (speedup 1.0000x reference)
import functools

import jax
import jax.numpy as jnp
from jax import lax
from jax.experimental import pallas as pl
from jax.experimental.pallas import tpu as pltpu
from jax.experimental.pallas import tpu_sc as plsc

N = 10000
E = 320000
D = 128
H = 128
DE = 16
X = 16
NC = 2
NS = 16
NW = NC * NS
EW = E // NW
WIN = 128
NFULL = EW // WIN
TAIL = EW - NFULL * WIN
NPT = N // NS
ZR = 125

_SC_PARAMS = pltpu.CompilerParams(use_tc_tiling_on_sc=False)


def _t1_body(h_ref, w1r_ref, w1c_ref, be1_ref, p_ref, q_ref):
    h = h_ref[...]
    p_ref[...] = (h @ w1r_ref[...] + be1_ref[...]).astype(jnp.bfloat16)
    q_ref[...] = (h @ w1c_ref[...]).astype(jnp.bfloat16)


def _t1(h, W1r, W1c, b_e1):
    BN = 2000
    return pl.pallas_call(
        _t1_body,
        grid=(N // BN,),
        in_specs=[
            pl.BlockSpec((BN, D), lambda i: (i, 0)),
            pl.BlockSpec((D, H), lambda i: (0, 0)),
            pl.BlockSpec((D, H), lambda i: (0, 0)),
            pl.BlockSpec((1, H), lambda i: (0, 0)),
        ],
        out_specs=[
            pl.BlockSpec((BN, H), lambda i: (i, 0)),
            pl.BlockSpec((BN, H), lambda i: (i, 0)),
        ],
        out_shape=[
            jax.ShapeDtypeStruct((N, H), jnp.bfloat16),
            jax.ShapeDtypeStruct((N, H), jnp.bfloat16),
        ],
    )(h, W1r, W1c, b_e1)


_NSLOT = 3


def _s1_body(row_hbm, col_hbm, p_hbm, q_hbm, cx_hbm, g_hbm, df_hbm, *refs):
    cid = lax.axis_index("c")
    sid = lax.axis_index("s")
    wbase = (sid * NC + cid) * EW

    slots = [refs[9 * j:9 * (j + 1)] for j in range(_NSLOT)]

    def start_idx(base, sl):
        ir, ic = sl[0], sl[1]
        si = sl[6]
        return (pltpu.async_copy(row_hbm.at[pl.ds(base, WIN)], ir, si),
                pltpu.async_copy(col_hbm.at[pl.ds(base, WIN)], ic, si))

    def start_gather(sl):
        ir, ic, gr, gc, cr, cc, _, sg, _ = sl
        return (pltpu.async_copy(p_hbm.at[ir], gr, sg),
                pltpu.async_copy(q_hbm.at[ic], gc, sg),
                pltpu.async_copy(cx_hbm.at[ir], cr, sg),
                pltpu.async_copy(cx_hbm.at[ic], cc, sg))

    def compute(sl):
        gr, gc, cr, cc = sl[2], sl[3], sl[4], sl[5]

        @plsc.parallel_loop(0, WIN, unroll=8)
        def _(r):
            for ch in range(H // 32):
                s = pl.ds(ch * 32, 32)
                gr[r, s] = gr[r, s] + gc[r, s]
            cr[r, :] = cr[r, :] - cc[r, :]

    def start_store(base, sl):
        gr, cr, ss = sl[2], sl[4], sl[8]
        return (pltpu.async_copy(gr, g_hbm.at[pl.ds(base, WIN)], ss),
                pltpu.async_copy(cr, df_hbm.at[pl.ds(base, WIN)], ss))

    def body(k, _):
        base = wbase + k * (_NSLOT * WIN)
        ha = [start_idx(base + j * WIN, slots[j]) for j in range(_NSLOT)]
        hg = []
        for j in range(_NSLOT):
            for hh in ha[j]:
                hh.wait()
            hg.append(start_gather(slots[j]))
        hs = []
        for j in range(_NSLOT):
            for hh in hg[j]:
                hh.wait()
            compute(slots[j])
            hs.append(start_store(base + j * WIN, slots[j]))
        for hj in hs:
            for hh in hj:
                hh.wait()
        return ()

    lax.fori_loop(0, NFULL // _NSLOT, body, ())

    ir, ic, gr, gc, cr, cc, si, sg, ss = slots[0]
    base = wbase + NFULL * WIN
    pltpu.sync_copy(row_hbm.at[pl.ds(base, TAIL)], ir.at[pl.ds(0, TAIL)])
    pltpu.sync_copy(col_hbm.at[pl.ds(base, TAIL)], ic.at[pl.ds(0, TAIL)])
    irs = ir.at[pl.ds(0, TAIL)]
    ics = ic.at[pl.ds(0, TAIL)]
    cp1 = pltpu.async_copy(p_hbm.at[irs], gr.at[pl.ds(0, TAIL)], sg)
    cp2 = pltpu.async_copy(q_hbm.at[ics], gc.at[pl.ds(0, TAIL)], sg)
    cp3 = pltpu.async_copy(cx_hbm.at[irs], cr.at[pl.ds(0, TAIL)], sg)
    cp4 = pltpu.async_copy(cx_hbm.at[ics], cc.at[pl.ds(0, TAIL)], sg)
    cp1.wait()
    cp2.wait()
    cp3.wait()
    cp4.wait()

    @plsc.parallel_loop(0, TAIL, unroll=8)
    def _(r):
        for ch in range(H // 32):
            s = pl.ds(ch * 32, 32)
            gr[r, s] = gr[r, s] + gc[r, s]
        cr[r, :] = cr[r, :] - cc[r, :]

    pltpu.sync_copy(gr.at[pl.ds(0, TAIL)], g_hbm.at[pl.ds(base, TAIL)])
    pltpu.sync_copy(cr.at[pl.ds(0, TAIL)], df_hbm.at[pl.ds(base, TAIL)])


def _s1(row, col, P, Q, CX):
    mesh = plsc.VectorSubcoreMesh(core_axis_name="c", subcore_axis_name="s")
    slot_scratch = []
    for _ in range(_NSLOT):
        slot_scratch += [
            pltpu.VMEM((WIN,), jnp.int32),
            pltpu.VMEM((WIN,), jnp.int32),
            pltpu.VMEM((WIN, H), jnp.bfloat16),
            pltpu.VMEM((WIN, H), jnp.bfloat16),
            pltpu.VMEM((WIN, X), jnp.float32),
            pltpu.VMEM((WIN, X), jnp.float32),
            pltpu.SemaphoreType.DMA,
            pltpu.SemaphoreType.DMA,
            pltpu.SemaphoreType.DMA,
        ]
    return pl.kernel(
        _s1_body,
        out_type=[
            jax.ShapeDtypeStruct((E, H), jnp.bfloat16),
            jax.ShapeDtypeStruct((E, X), jnp.float32),
        ],
        mesh=mesh,
        compiler_params=_SC_PARAMS,
        scratch_types=slot_scratch,
    )(row, col, P, Q, CX)


def _t2_body(g_ref, df_ref, ea_ref, w1a_ref, w1rad_ref, we2_ref,
             be2_ref, wc1_ref, bc1_ref, wc2_ref, f_ref, fx_ref):
    diff = df_ref[:, :3]
    radial = jnp.sum(diff * diff, axis=1, keepdims=True)
    m_in = g_ref[...].astype(jnp.float32) + radial * w1rad_ref[...] \
        + ea_ref[...] @ w1a_ref[...]
    m = jax.nn.silu(m_in)
    ef = jax.nn.silu(m @ we2_ref[...] + be2_ref[...])
    c1 = jax.nn.silu(ef @ wc1_ref[...] + bc1_ref[...])
    phi = c1 @ wc2_ref[...]
    bt = ef.shape[0]
    f_ref[...] = ef
    fx_ref[...] = jnp.concatenate(
        [diff * phi, jnp.ones((bt, 1), jnp.float32),
         jnp.zeros((bt, X - 4), jnp.float32)], axis=1)


def _t2(G, DF, edge_attr, W1a, w1rad, W_e2, b_e2, W_c1, b_c1, W_c2):
    BT = 1280
    return pl.pallas_call(
        _t2_body,
        grid=(E // BT,),
        in_specs=[
            pl.BlockSpec((BT, H), lambda i: (i, 0)),
            pl.BlockSpec((BT, X), lambda i: (i, 0)),
            pl.BlockSpec((BT, DE), lambda i: (i, 0)),
            pl.BlockSpec((DE, H), lambda i: (0, 0)),
            pl.BlockSpec((1, H), lambda i: (0, 0)),
            pl.BlockSpec((H, H), lambda i: (0, 0)),
            pl.BlockSpec((1, H), lambda i: (0, 0)),
            pl.BlockSpec((H, H), lambda i: (0, 0)),
            pl.BlockSpec((1, H), lambda i: (0, 0)),
            pl.BlockSpec((H, 1), lambda i: (0, 0)),
        ],
        out_specs=[
            pl.BlockSpec((BT, H), lambda i: (i, 0)),
            pl.BlockSpec((BT, X), lambda i: (i, 0)),
        ],
        out_shape=[
            jax.ShapeDtypeStruct((E, H), jnp.float32),
            jax.ShapeDtypeStruct((E, X), jnp.float32),
        ],
    )(G, DF, edge_attr, W1a, w1rad, W_e2, b_e2, W_c1, b_c1, W_c2)


def _s2_body(row_hbm, f_hbm, fx_hbm, acc2_hbm, accx2_hbm,
             ib, fb, fxb, zb, zxb, acc, accx, sem):
    cid = lax.axis_index("c")
    sid = lax.axis_index("s")
    wbase = (sid * NC + cid) * EW

    zero16 = jnp.zeros((16,), jnp.float32)

    def zrow(r, _):
        for ch in range(H // 16):
            zb[r, pl.ds(ch * 16, 16)] = zero16
        zxb[r, :] = zero16
        return ()

    lax.fori_loop(0, ZR, zrow, ())

    def zchunk(k, _):
        pltpu.sync_copy(zb, acc.at[pl.ds(sid * NPT + k * ZR, ZR)])
        pltpu.sync_copy(zxb, accx.at[pl.ds(sid * NPT + k * ZR, ZR)])
        return ()

    lax.fori_loop(0, NPT // ZR, zchunk, ())
    plsc.subcore_barrier()

    def do_win(base, size):
        pltpu.sync_copy(row_hbm.at[pl.ds(base, size)], ib.at[pl.ds(0, size)])
        ibs = ib.at[pl.ds(0, size)] if size != WIN else ib
        pltpu.sync_copy(f_hbm.at[pl.ds(base, size)], fb.at[pl.ds(0, size)])
        pltpu.sync_copy(fx_hbm.at[pl.ds(base, size)], fxb.at[pl.ds(0, size)])
        pltpu.sync_copy(fb.at[pl.ds(0, size)], acc.at[ibs], add=True)
        pltpu.sync_copy(fxb.at[pl.ds(0, size)], accx.at[ibs], add=True)

    def win(wi, _):
        do_win(wbase + wi * WIN, WIN)
        return ()

    lax.fori_loop(0, NFULL, win, ())
    do_win(wbase + NFULL * WIN, TAIL)

    plsc.subcore_barrier()

    def flush(k, _):
        off = sid * NPT + k * ZR
        pltpu.sync_copy(acc.at[pl.ds(off, ZR)], acc2_hbm.at[cid, pl.ds(off, ZR)])
        pltpu.sync_copy(accx.at[pl.ds(off, ZR)],
                        accx2_hbm.at[cid, pl.ds(off, ZR)])
        return ()

    lax.fori_loop(0, NPT // ZR, flush, ())


def _s2(row, F, FX):
    mesh = plsc.VectorSubcoreMesh(core_axis_name="c", subcore_axis_name="s")
    return pl.kernel(
        _s2_body,
        out_type=[
            jax.ShapeDtypeStruct((NC, N, H), jnp.float32),
            jax.ShapeDtypeStruct((NC, N, X), jnp.float32),
        ],
        mesh=mesh,
        compiler_params=_SC_PARAMS,
        scratch_types=[
            pltpu.VMEM((WIN,), jnp.int32),
            pltpu.VMEM((WIN, H), jnp.float32),
            pltpu.VMEM((WIN, X), jnp.float32),
            pltpu.VMEM((ZR, H), jnp.float32),
            pltpu.VMEM((ZR, X), jnp.float32),
            pltpu.VMEM_SHARED((N, H), jnp.float32),
            pltpu.VMEM_SHARED((N, X), jnp.float32),
            pltpu.SemaphoreType.DMA,
        ],
    )(row, F, FX)


def _t3_body(acc2_ref, accx2_ref, h_ref, c_ref, wn1h_ref, wn1a_ref, bn1_ref,
             wn2_ref, bn2_ref, ho_ref, co_ref):
    agg_h = acc2_ref[0] + acc2_ref[1]
    accx = accx2_ref[0] + accx2_ref[1]
    sums = accx[:, :3]
    cnt = accx[:, 3:4]
    h = h_ref[...]
    t = jax.nn.silu(h @ wn1h_ref[...] + agg_h @ wn1a_ref[...] + bn1_ref[...])
    ho_ref[...] = h + t @ wn2_ref[...] + bn2_ref[...]
    co_ref[...] = c_ref[...] + sums / jnp.maximum(cnt, 1.0)


def _t3(ACC2, ACCX2, h, coord, Wn1h, Wn1a, b_n1, W_n2, b_n2):
    BN = 2000
    return pl.pallas_call(
        _t3_body,
        grid=(N // BN,),
        in_specs=[
            pl.BlockSpec((NC, BN, H), lambda i: (0, i, 0)),
            pl.BlockSpec((NC, BN, X), lambda i: (0, i, 0)),
            pl.BlockSpec((BN, D), lambda i: (i, 0)),
            pl.BlockSpec((BN, 3), lambda i: (i, 0)),
            pl.BlockSpec((D, H), lambda i: (0, 0)),
            pl.BlockSpec((H, H), lambda i: (0, 0)),
            pl.BlockSpec((1, H), lambda i: (0, 0)),
            pl.BlockSpec((H, D), lambda i: (0, 0)),
            pl.BlockSpec((1, D), lambda i: (0, 0)),
        ],
        out_specs=[
            pl.BlockSpec((BN, D), lambda i: (i, 0)),
            pl.BlockSpec((BN, 3), lambda i: (i, 0)),
        ],
        out_shape=[
            jax.ShapeDtypeStruct((N, D), jnp.float32),
            jax.ShapeDtypeStruct((N, 3), jnp.float32),
        ],
    )(ACC2, ACCX2, h, coord, Wn1h, Wn1a, b_n1, W_n2, b_n2)


@jax.jit
def kernel(h, edge_index, coord, edge_attr, W_e1, b_e1, W_e2, b_e2,
           W_n1, b_n1, W_n2, b_n2, W_c1, b_c1, W_c2):
    row = edge_index[0]
    col = edge_index[1]

    W1r = W_e1[:D]
    W1c = W_e1[D:2 * D]
    w1rad = W_e1[2 * D:2 * D + 1]
    W1a = W_e1[2 * D + 1:]

    CX = jnp.pad(coord, ((0, 0), (0, X - 3)))

    P, Q = _t1(h, W1r, W1c, b_e1.reshape(1, H))
    G, DF = _s1(row, col, P, Q, CX)
    F, FX = _t2(G, DF, edge_attr, W1a, w1rad, W_e2, b_e2.reshape(1, H),
                W_c1, b_c1.reshape(1, H), W_c2)
    ACC2, ACCX2 = _s2(row, F, FX)
    h_out, coord_out = _t3(ACC2, ACCX2, h, coord, W_n1[:D], W_n1[D:],
                           b_n1.reshape(1, H), W_n2, b_n2.reshape(1, D))
    return (h_out, coord_out, edge_attr)


# speedup vs baseline: 1.1789x; 1.1789x over previous
import functools

import jax
import jax.numpy as jnp
from jax import lax
from jax.experimental import pallas as pl
from jax.experimental.pallas import tpu as pltpu
from jax.experimental.pallas import tpu_sc as plsc

N = 10000
E = 320000
D = 128
H = 128
DE = 16
X = 16
NC = 2
NS = 16
NW = NC * NS
EW = E // NW
WIN = 128
NFULL = EW // WIN
TAIL = EW - NFULL * WIN
NPT = N // NS
ZR = 125

_SC_PARAMS = pltpu.CompilerParams(use_tc_tiling_on_sc=False)


def _t1_body(h_ref, w1r_ref, w1c_ref, be1_ref, p_ref, q_ref):
    h = h_ref[...]
    p_ref[...] = h @ w1r_ref[...] + be1_ref[...]
    q_ref[...] = h @ w1c_ref[...]


def _t1(h, W1r, W1c, b_e1):
    BN = 2000
    return pl.pallas_call(
        _t1_body,
        grid=(N // BN,),
        in_specs=[
            pl.BlockSpec((BN, D), lambda i: (i, 0)),
            pl.BlockSpec((D, H), lambda i: (0, 0)),
            pl.BlockSpec((D, H), lambda i: (0, 0)),
            pl.BlockSpec((1, H), lambda i: (0, 0)),
        ],
        out_specs=[
            pl.BlockSpec((BN, H), lambda i: (i, 0)),
            pl.BlockSpec((BN, H), lambda i: (i, 0)),
        ],
        out_shape=[
            jax.ShapeDtypeStruct((N, H), jnp.float32),
            jax.ShapeDtypeStruct((N, H), jnp.float32),
        ],
    )(h, W1r, W1c, b_e1)


_NSLOT = 3


def _s1_body(row_hbm, col_hbm, p_hbm, q_hbm, cx_hbm, g_hbm, df_hbm, *refs):
    cid = lax.axis_index("c")
    sid = lax.axis_index("s")
    wbase = (sid * NC + cid) * EW

    slots = [refs[9 * j:9 * (j + 1)] for j in range(_NSLOT)]

    def start_idx(base, sl):
        ir, ic = sl[0], sl[1]
        si = sl[6]
        return (pltpu.async_copy(row_hbm.at[pl.ds(base, WIN)], ir, si),
                pltpu.async_copy(col_hbm.at[pl.ds(base, WIN)], ic, si))

    def start_gather(sl):
        ir, ic, gr, gc, cr, cc, _, sg, _ = sl
        return (pltpu.async_copy(p_hbm.at[ir], gr, sg),
                pltpu.async_copy(q_hbm.at[ic], gc, sg),
                pltpu.async_copy(cx_hbm.at[ir], cr, sg),
                pltpu.async_copy(cx_hbm.at[ic], cc, sg))

    def compute(sl):
        gr, gc, cr, cc = sl[2], sl[3], sl[4], sl[5]

        @plsc.parallel_loop(0, WIN, unroll=8)
        def _(r):
            for ch in range(H // 16):
                s = pl.ds(ch * 16, 16)
                gr[r, s] = gr[r, s] + gc[r, s]
            cr[r, :] = cr[r, :] - cc[r, :]

    def start_store(base, sl):
        gr, cr, ss = sl[2], sl[4], sl[8]
        return (pltpu.async_copy(gr, g_hbm.at[pl.ds(base, WIN)], ss),
                pltpu.async_copy(cr, df_hbm.at[pl.ds(base, WIN)], ss))

    def body(k, _):
        base = wbase + k * (_NSLOT * WIN)
        ha = [start_idx(base + j * WIN, slots[j]) for j in range(_NSLOT)]
        hg = []
        for j in range(_NSLOT):
            for hh in ha[j]:
                hh.wait()
            hg.append(start_gather(slots[j]))
        hs = []
        for j in range(_NSLOT):
            for hh in hg[j]:
                hh.wait()
            compute(slots[j])
            hs.append(start_store(base + j * WIN, slots[j]))
        for hj in hs:
            for hh in hj:
                hh.wait()
        return ()

    lax.fori_loop(0, NFULL // _NSLOT, body, ())

    ir, ic, gr, gc, cr, cc, si, sg, ss = slots[0]
    base = wbase + NFULL * WIN
    pltpu.sync_copy(row_hbm.at[pl.ds(base, TAIL)], ir.at[pl.ds(0, TAIL)])
    pltpu.sync_copy(col_hbm.at[pl.ds(base, TAIL)], ic.at[pl.ds(0, TAIL)])
    irs = ir.at[pl.ds(0, TAIL)]
    ics = ic.at[pl.ds(0, TAIL)]
    cp1 = pltpu.async_copy(p_hbm.at[irs], gr.at[pl.ds(0, TAIL)], sg)
    cp2 = pltpu.async_copy(q_hbm.at[ics], gc.at[pl.ds(0, TAIL)], sg)
    cp3 = pltpu.async_copy(cx_hbm.at[irs], cr.at[pl.ds(0, TAIL)], sg)
    cp4 = pltpu.async_copy(cx_hbm.at[ics], cc.at[pl.ds(0, TAIL)], sg)
    cp1.wait()
    cp2.wait()
    cp3.wait()
    cp4.wait()

    @plsc.parallel_loop(0, TAIL, unroll=8)
    def _(r):
        for ch in range(H // 16):
            s = pl.ds(ch * 16, 16)
            gr[r, s] = gr[r, s] + gc[r, s]
        cr[r, :] = cr[r, :] - cc[r, :]

    pltpu.sync_copy(gr.at[pl.ds(0, TAIL)], g_hbm.at[pl.ds(base, TAIL)])
    pltpu.sync_copy(cr.at[pl.ds(0, TAIL)], df_hbm.at[pl.ds(base, TAIL)])


def _s1(row, col, P, Q, CX):
    mesh = plsc.VectorSubcoreMesh(core_axis_name="c", subcore_axis_name="s")
    slot_scratch = []
    for _ in range(_NSLOT):
        slot_scratch += [
            pltpu.VMEM((WIN,), jnp.int32),
            pltpu.VMEM((WIN,), jnp.int32),
            pltpu.VMEM((WIN, H), jnp.float32),
            pltpu.VMEM((WIN, H), jnp.float32),
            pltpu.VMEM((WIN, X), jnp.float32),
            pltpu.VMEM((WIN, X), jnp.float32),
            pltpu.SemaphoreType.DMA,
            pltpu.SemaphoreType.DMA,
            pltpu.SemaphoreType.DMA,
        ]
    return pl.kernel(
        _s1_body,
        out_type=[
            jax.ShapeDtypeStruct((E, H), jnp.float32),
            jax.ShapeDtypeStruct((E, X), jnp.float32),
        ],
        mesh=mesh,
        compiler_params=_SC_PARAMS,
        scratch_types=slot_scratch,
    )(row, col, P, Q, CX)


def _t2_body(g_ref, df_ref, ea_ref, w1a_ref, w1rad_ref, we2_ref,
             be2_ref, wc1_ref, bc1_ref, wc2_ref, f_ref, fx_ref):
    diff = df_ref[:, :3]
    radial = jnp.sum(diff * diff, axis=1, keepdims=True)
    m_in = g_ref[...] + radial * w1rad_ref[...] + ea_ref[...] @ w1a_ref[...]
    m = jax.nn.silu(m_in)
    ef = jax.nn.silu(m @ we2_ref[...] + be2_ref[...])
    c1 = jax.nn.silu(ef @ wc1_ref[...] + bc1_ref[...])
    phi = c1 @ wc2_ref[...]
    bt = ef.shape[0]
    f_ref[...] = ef
    fx_ref[...] = jnp.concatenate(
        [diff * phi, jnp.ones((bt, 1), jnp.float32),
         jnp.zeros((bt, X - 4), jnp.float32)], axis=1)


def _t2(G, DF, edge_attr, W1a, w1rad, W_e2, b_e2, W_c1, b_c1, W_c2):
    BT = 1280
    return pl.pallas_call(
        _t2_body,
        grid=(E // BT,),
        in_specs=[
            pl.BlockSpec((BT, H), lambda i: (i, 0)),
            pl.BlockSpec((BT, X), lambda i: (i, 0)),
            pl.BlockSpec((BT, DE), lambda i: (i, 0)),
            pl.BlockSpec((DE, H), lambda i: (0, 0)),
            pl.BlockSpec((1, H), lambda i: (0, 0)),
            pl.BlockSpec((H, H), lambda i: (0, 0)),
            pl.BlockSpec((1, H), lambda i: (0, 0)),
            pl.BlockSpec((H, H), lambda i: (0, 0)),
            pl.BlockSpec((1, H), lambda i: (0, 0)),
            pl.BlockSpec((H, 1), lambda i: (0, 0)),
        ],
        out_specs=[
            pl.BlockSpec((BT, H), lambda i: (i, 0)),
            pl.BlockSpec((BT, X), lambda i: (i, 0)),
        ],
        out_shape=[
            jax.ShapeDtypeStruct((E, H), jnp.float32),
            jax.ShapeDtypeStruct((E, X), jnp.float32),
        ],
    )(G, DF, edge_attr, W1a, w1rad, W_e2, b_e2, W_c1, b_c1, W_c2)


def _s2_body(row_hbm, f_hbm, fx_hbm, acc2_hbm, accx2_hbm,
             ib, fb, fxb, zb, zxb, acc, accx, sem):
    cid = lax.axis_index("c")
    sid = lax.axis_index("s")
    wbase = (sid * NC + cid) * EW

    zero16 = jnp.zeros((16,), jnp.float32)

    def zrow(r, _):
        for ch in range(H // 16):
            zb[r, pl.ds(ch * 16, 16)] = zero16
        zxb[r, :] = zero16
        return ()

    lax.fori_loop(0, ZR, zrow, ())

    def zchunk(k, _):
        pltpu.sync_copy(zb, acc.at[pl.ds(sid * NPT + k * ZR, ZR)])
        pltpu.sync_copy(zxb, accx.at[pl.ds(sid * NPT + k * ZR, ZR)])
        return ()

    lax.fori_loop(0, NPT // ZR, zchunk, ())
    plsc.subcore_barrier()

    def do_win(base, size):
        pltpu.sync_copy(row_hbm.at[pl.ds(base, size)], ib.at[pl.ds(0, size)])
        ibs = ib.at[pl.ds(0, size)] if size != WIN else ib
        pltpu.sync_copy(f_hbm.at[pl.ds(base, size)], fb.at[pl.ds(0, size)])
        pltpu.sync_copy(fx_hbm.at[pl.ds(base, size)], fxb.at[pl.ds(0, size)])
        pltpu.sync_copy(fb.at[pl.ds(0, size)], acc.at[ibs], add=True)
        pltpu.sync_copy(fxb.at[pl.ds(0, size)], accx.at[ibs], add=True)

    def win(wi, _):
        do_win(wbase + wi * WIN, WIN)
        return ()

    lax.fori_loop(0, NFULL, win, ())
    do_win(wbase + NFULL * WIN, TAIL)

    plsc.subcore_barrier()

    def flush(k, _):
        off = sid * NPT + k * ZR
        pltpu.sync_copy(acc.at[pl.ds(off, ZR)], acc2_hbm.at[cid, pl.ds(off, ZR)])
        pltpu.sync_copy(accx.at[pl.ds(off, ZR)],
                        accx2_hbm.at[cid, pl.ds(off, ZR)])
        return ()

    lax.fori_loop(0, NPT // ZR, flush, ())


def _s2(row, F, FX):
    mesh = plsc.VectorSubcoreMesh(core_axis_name="c", subcore_axis_name="s")
    return pl.kernel(
        _s2_body,
        out_type=[
            jax.ShapeDtypeStruct((NC, N, H), jnp.float32),
            jax.ShapeDtypeStruct((NC, N, X), jnp.float32),
        ],
        mesh=mesh,
        compiler_params=_SC_PARAMS,
        scratch_types=[
            pltpu.VMEM((WIN,), jnp.int32),
            pltpu.VMEM((WIN, H), jnp.float32),
            pltpu.VMEM((WIN, X), jnp.float32),
            pltpu.VMEM((ZR, H), jnp.float32),
            pltpu.VMEM((ZR, X), jnp.float32),
            pltpu.VMEM_SHARED((N, H), jnp.float32),
            pltpu.VMEM_SHARED((N, X), jnp.float32),
            pltpu.SemaphoreType.DMA,
        ],
    )(row, F, FX)


def _t3_body(acc2_ref, accx2_ref, h_ref, c_ref, wn1h_ref, wn1a_ref, bn1_ref,
             wn2_ref, bn2_ref, ho_ref, co_ref):
    agg_h = acc2_ref[0] + acc2_ref[1]
    accx = accx2_ref[0] + accx2_ref[1]
    sums = accx[:, :3]
    cnt = accx[:, 3:4]
    h = h_ref[...]
    t = jax.nn.silu(h @ wn1h_ref[...] + agg_h @ wn1a_ref[...] + bn1_ref[...])
    ho_ref[...] = h + t @ wn2_ref[...] + bn2_ref[...]
    co_ref[...] = c_ref[...] + sums / jnp.maximum(cnt, 1.0)


def _t3(ACC2, ACCX2, h, coord, Wn1h, Wn1a, b_n1, W_n2, b_n2):
    BN = 2000
    return pl.pallas_call(
        _t3_body,
        grid=(N // BN,),
        in_specs=[
            pl.BlockSpec((NC, BN, H), lambda i: (0, i, 0)),
            pl.BlockSpec((NC, BN, X), lambda i: (0, i, 0)),
            pl.BlockSpec((BN, D), lambda i: (i, 0)),
            pl.BlockSpec((BN, 3), lambda i: (i, 0)),
            pl.BlockSpec((D, H), lambda i: (0, 0)),
            pl.BlockSpec((H, H), lambda i: (0, 0)),
            pl.BlockSpec((1, H), lambda i: (0, 0)),
            pl.BlockSpec((H, D), lambda i: (0, 0)),
            pl.BlockSpec((1, D), lambda i: (0, 0)),
        ],
        out_specs=[
            pl.BlockSpec((BN, D), lambda i: (i, 0)),
            pl.BlockSpec((BN, 3), lambda i: (i, 0)),
        ],
        out_shape=[
            jax.ShapeDtypeStruct((N, D), jnp.float32),
            jax.ShapeDtypeStruct((N, 3), jnp.float32),
        ],
    )(ACC2, ACCX2, h, coord, Wn1h, Wn1a, b_n1, W_n2, b_n2)


@jax.jit
def kernel(h, edge_index, coord, edge_attr, W_e1, b_e1, W_e2, b_e2,
           W_n1, b_n1, W_n2, b_n2, W_c1, b_c1, W_c2):
    row = edge_index[0]
    col = edge_index[1]

    W1r = W_e1[:D]
    W1c = W_e1[D:2 * D]
    w1rad = W_e1[2 * D:2 * D + 1]
    W1a = W_e1[2 * D + 1:]

    CX = jnp.pad(coord, ((0, 0), (0, X - 3)))

    P, Q = _t1(h, W1r, W1c, b_e1.reshape(1, H))
    G, DF = _s1(row, col, P, Q, CX)
    F, FX = _t2(G, DF, edge_attr, W1a, w1rad, W_e2, b_e2.reshape(1, H),
                W_c1, b_c1.reshape(1, H), W_c2)
    ACC2, ACCX2 = _s2(row, F, FX)
    h_out, coord_out = _t3(ACC2, ACCX2, h, coord, W_n1[:D], W_n1[D:],
                           b_n1.reshape(1, H), W_n2, b_n2.reshape(1, D))
    return (h_out, coord_out, edge_attr)


# speedup vs baseline: 1.3276x; 1.1261x over previous
import functools

import jax
import jax.numpy as jnp
from jax import lax
from jax.experimental import pallas as pl
from jax.experimental.pallas import tpu as pltpu
from jax.experimental.pallas import tpu_sc as plsc

N = 10000
E = 320000
D = 128
H = 128
DE = 16
X = 16
NC = 2
NS = 16
NW = NC * NS
CHUNKS = 2
EC = E // CHUNKS
EW = EC // NW
WIN = 128
NFULL = EW // WIN
TAIL = EW - NFULL * WIN
NPT = N // NS
ZR = 125

_SC_PARAMS = pltpu.CompilerParams(use_tc_tiling_on_sc=False)


def _t1_body(h_ref, w1r_ref, w1c_ref, be1_ref, p_ref, q_ref):
    h = h_ref[...]
    p_ref[...] = h @ w1r_ref[...] + be1_ref[...]
    q_ref[...] = h @ w1c_ref[...]


def _t1(h, W1r, W1c, b_e1):
    BN = 2000
    return pl.pallas_call(
        _t1_body,
        grid=(N // BN,),
        in_specs=[
            pl.BlockSpec((BN, D), lambda i: (i, 0)),
            pl.BlockSpec((D, H), lambda i: (0, 0)),
            pl.BlockSpec((D, H), lambda i: (0, 0)),
            pl.BlockSpec((1, H), lambda i: (0, 0)),
        ],
        out_specs=[
            pl.BlockSpec((BN, H), lambda i: (i, 0)),
            pl.BlockSpec((BN, H), lambda i: (i, 0)),
        ],
        out_shape=[
            jax.ShapeDtypeStruct((N, H), jnp.float32),
            jax.ShapeDtypeStruct((N, H), jnp.float32),
        ],
    )(h, W1r, W1c, b_e1)


_NSLOT = 3


def _s1_body(row_hbm, col_hbm, p_hbm, q_hbm, cx_hbm, g_hbm, df_hbm, *refs):
    cid = lax.axis_index("c")
    sid = lax.axis_index("s")
    wbase = (sid * NC + cid) * EW

    slots = [refs[9 * j:9 * (j + 1)] for j in range(_NSLOT)]

    def start_idx(base, sl):
        ir, ic = sl[0], sl[1]
        si = sl[6]
        return (pltpu.async_copy(row_hbm.at[pl.ds(base, WIN)], ir, si),
                pltpu.async_copy(col_hbm.at[pl.ds(base, WIN)], ic, si))

    def start_gather(sl):
        ir, ic, gr, gc, cr, cc, _, sg, _ = sl
        return (pltpu.async_copy(p_hbm.at[ir], gr, sg),
                pltpu.async_copy(q_hbm.at[ic], gc, sg),
                pltpu.async_copy(cx_hbm.at[ir], cr, sg),
                pltpu.async_copy(cx_hbm.at[ic], cc, sg))

    def compute(sl):
        gr, gc, cr, cc = sl[2], sl[3], sl[4], sl[5]

        @plsc.parallel_loop(0, WIN, unroll=8)
        def _(r):
            for ch in range(H // 16):
                s = pl.ds(ch * 16, 16)
                gr[r, s] = gr[r, s] + gc[r, s]
            cr[r, :] = cr[r, :] - cc[r, :]

    def start_store(base, sl):
        gr, cr, ss = sl[2], sl[4], sl[8]
        return (pltpu.async_copy(gr, g_hbm.at[pl.ds(base, WIN)], ss),
                pltpu.async_copy(cr, df_hbm.at[pl.ds(base, WIN)], ss))

    def body(k, _):
        base = wbase + k * (_NSLOT * WIN)
        ha = [start_idx(base + j * WIN, slots[j]) for j in range(_NSLOT)]
        hg = []
        for j in range(_NSLOT):
            for hh in ha[j]:
                hh.wait()
            hg.append(start_gather(slots[j]))
        hs = []
        for j in range(_NSLOT):
            for hh in hg[j]:
                hh.wait()
            compute(slots[j])
            hs.append(start_store(base + j * WIN, slots[j]))
        for hj in hs:
            for hh in hj:
                hh.wait()
        return ()

    lax.fori_loop(0, NFULL // _NSLOT, body, ())

    ir, ic, gr, gc, cr, cc, si, sg, ss = slots[0]
    base = wbase + NFULL * WIN
    pltpu.sync_copy(row_hbm.at[pl.ds(base, TAIL)], ir.at[pl.ds(0, TAIL)])
    pltpu.sync_copy(col_hbm.at[pl.ds(base, TAIL)], ic.at[pl.ds(0, TAIL)])
    irs = ir.at[pl.ds(0, TAIL)]
    ics = ic.at[pl.ds(0, TAIL)]
    cp1 = pltpu.async_copy(p_hbm.at[irs], gr.at[pl.ds(0, TAIL)], sg)
    cp2 = pltpu.async_copy(q_hbm.at[ics], gc.at[pl.ds(0, TAIL)], sg)
    cp3 = pltpu.async_copy(cx_hbm.at[irs], cr.at[pl.ds(0, TAIL)], sg)
    cp4 = pltpu.async_copy(cx_hbm.at[ics], cc.at[pl.ds(0, TAIL)], sg)
    cp1.wait()
    cp2.wait()
    cp3.wait()
    cp4.wait()

    @plsc.parallel_loop(0, TAIL, unroll=8)
    def _(r):
        for ch in range(H // 16):
            s = pl.ds(ch * 16, 16)
            gr[r, s] = gr[r, s] + gc[r, s]
        cr[r, :] = cr[r, :] - cc[r, :]

    pltpu.sync_copy(gr.at[pl.ds(0, TAIL)], g_hbm.at[pl.ds(base, TAIL)])
    pltpu.sync_copy(cr.at[pl.ds(0, TAIL)], df_hbm.at[pl.ds(base, TAIL)])


def _s1(row, col, P, Q, CX):
    mesh = plsc.VectorSubcoreMesh(core_axis_name="c", subcore_axis_name="s")
    slot_scratch = []
    for _ in range(_NSLOT):
        slot_scratch += [
            pltpu.VMEM((WIN,), jnp.int32),
            pltpu.VMEM((WIN,), jnp.int32),
            pltpu.VMEM((WIN, H), jnp.float32),
            pltpu.VMEM((WIN, H), jnp.float32),
            pltpu.VMEM((WIN, X), jnp.float32),
            pltpu.VMEM((WIN, X), jnp.float32),
            pltpu.SemaphoreType.DMA,
            pltpu.SemaphoreType.DMA,
            pltpu.SemaphoreType.DMA,
        ]
    return pl.kernel(
        _s1_body,
        out_type=[
            jax.ShapeDtypeStruct((EC, H), jnp.float32),
            jax.ShapeDtypeStruct((EC, X), jnp.float32),
        ],
        name="s1_gather",
        mesh=mesh,
        compiler_params=_SC_PARAMS,
        scratch_types=slot_scratch,
    )(row, col, P, Q, CX)


def _t2_body(g_ref, df_ref, ea_ref, w1a_ref, w1rad_ref, we2_ref,
             be2_ref, wc1_ref, bc1_ref, wc2_ref, f_ref, fx_ref):
    diff = df_ref[:, :3]
    radial = jnp.sum(diff * diff, axis=1, keepdims=True)
    m_in = g_ref[...] + radial * w1rad_ref[...] + ea_ref[...] @ w1a_ref[...]
    m = jax.nn.silu(m_in)
    ef = jax.nn.silu(m @ we2_ref[...] + be2_ref[...])
    c1 = jax.nn.silu(ef @ wc1_ref[...] + bc1_ref[...])
    phi = c1 @ wc2_ref[...]
    bt = ef.shape[0]
    f_ref[...] = ef
    fx_ref[...] = jnp.concatenate(
        [diff * phi, jnp.ones((bt, 1), jnp.float32),
         jnp.zeros((bt, X - 4), jnp.float32)], axis=1)


def _t2(G, DF, edge_attr, W1a, w1rad, W_e2, b_e2, W_c1, b_c1, W_c2):
    BT = 1280
    return pl.pallas_call(
        _t2_body,
        grid=(EC // BT,),
        in_specs=[
            pl.BlockSpec((BT, H), lambda i: (i, 0)),
            pl.BlockSpec((BT, X), lambda i: (i, 0)),
            pl.BlockSpec((BT, DE), lambda i: (i, 0)),
            pl.BlockSpec((DE, H), lambda i: (0, 0)),
            pl.BlockSpec((1, H), lambda i: (0, 0)),
            pl.BlockSpec((H, H), lambda i: (0, 0)),
            pl.BlockSpec((1, H), lambda i: (0, 0)),
            pl.BlockSpec((H, H), lambda i: (0, 0)),
            pl.BlockSpec((1, H), lambda i: (0, 0)),
            pl.BlockSpec((H, 1), lambda i: (0, 0)),
        ],
        out_specs=[
            pl.BlockSpec((BT, H), lambda i: (i, 0)),
            pl.BlockSpec((BT, X), lambda i: (i, 0)),
        ],
        out_shape=[
            jax.ShapeDtypeStruct((EC, H), jnp.float32),
            jax.ShapeDtypeStruct((EC, X), jnp.float32),
        ],
    )(G, DF, edge_attr, W1a, w1rad, W_e2, b_e2, W_c1, b_c1, W_c2)


def _s2_body(row_hbm, f_hbm, fx_hbm, acc2_hbm, accx2_hbm,
             ib, fb, fxb, zb, zxb, acc, accx, sem):
    cid = lax.axis_index("c")
    sid = lax.axis_index("s")
    wbase = (sid * NC + cid) * EW

    zero16 = jnp.zeros((16,), jnp.float32)

    def zrow(r, _):
        for ch in range(H // 16):
            zb[r, pl.ds(ch * 16, 16)] = zero16
        zxb[r, :] = zero16
        return ()

    lax.fori_loop(0, ZR, zrow, ())

    def zchunk(k, _):
        pltpu.sync_copy(zb, acc.at[pl.ds(sid * NPT + k * ZR, ZR)])
        pltpu.sync_copy(zxb, accx.at[pl.ds(sid * NPT + k * ZR, ZR)])
        return ()

    lax.fori_loop(0, NPT // ZR, zchunk, ())
    plsc.subcore_barrier()

    def do_win(base, size):
        pltpu.sync_copy(row_hbm.at[pl.ds(base, size)], ib.at[pl.ds(0, size)])
        ibs = ib.at[pl.ds(0, size)] if size != WIN else ib
        pltpu.sync_copy(f_hbm.at[pl.ds(base, size)], fb.at[pl.ds(0, size)])
        pltpu.sync_copy(fx_hbm.at[pl.ds(base, size)], fxb.at[pl.ds(0, size)])
        pltpu.sync_copy(fb.at[pl.ds(0, size)], acc.at[ibs], add=True)
        pltpu.sync_copy(fxb.at[pl.ds(0, size)], accx.at[ibs], add=True)

    def win(wi, _):
        do_win(wbase + wi * WIN, WIN)
        return ()

    lax.fori_loop(0, NFULL, win, ())
    do_win(wbase + NFULL * WIN, TAIL)

    plsc.subcore_barrier()

    def flush(k, _):
        off = sid * NPT + k * ZR
        pltpu.sync_copy(acc.at[pl.ds(off, ZR)], acc2_hbm.at[cid, pl.ds(off, ZR)])
        pltpu.sync_copy(accx.at[pl.ds(off, ZR)],
                        accx2_hbm.at[cid, pl.ds(off, ZR)])
        return ()

    lax.fori_loop(0, NPT // ZR, flush, ())


def _s2(row, F, FX):
    mesh = plsc.VectorSubcoreMesh(core_axis_name="c", subcore_axis_name="s")
    return pl.kernel(
        _s2_body,
        out_type=[
            jax.ShapeDtypeStruct((NC, N, H), jnp.float32),
            jax.ShapeDtypeStruct((NC, N, X), jnp.float32),
        ],
        name="s2_scatter",
        mesh=mesh,
        compiler_params=_SC_PARAMS,
        scratch_types=[
            pltpu.VMEM((WIN,), jnp.int32),
            pltpu.VMEM((WIN, H), jnp.float32),
            pltpu.VMEM((WIN, X), jnp.float32),
            pltpu.VMEM((ZR, H), jnp.float32),
            pltpu.VMEM((ZR, X), jnp.float32),
            pltpu.VMEM_SHARED((N, H), jnp.float32),
            pltpu.VMEM_SHARED((N, X), jnp.float32),
            pltpu.SemaphoreType.DMA,
        ],
    )(row, F, FX)


def _t3_body(acc2a_ref, accxa_ref, acc2b_ref, accxb_ref, h_ref, c_ref,
             wn1h_ref, wn1a_ref, bn1_ref, wn2_ref, bn2_ref, ho_ref, co_ref):
    agg_h = (acc2a_ref[0] + acc2a_ref[1]) + (acc2b_ref[0] + acc2b_ref[1])
    accx = (accxa_ref[0] + accxa_ref[1]) + (accxb_ref[0] + accxb_ref[1])
    sums = accx[:, :3]
    cnt = accx[:, 3:4]
    h = h_ref[...]
    t = jax.nn.silu(h @ wn1h_ref[...] + agg_h @ wn1a_ref[...] + bn1_ref[...])
    ho_ref[...] = h + t @ wn2_ref[...] + bn2_ref[...]
    co_ref[...] = c_ref[...] + sums / jnp.maximum(cnt, 1.0)


def _t3(ACC2a, ACCXa, ACC2b, ACCXb, h, coord, Wn1h, Wn1a, b_n1, W_n2, b_n2):
    BN = 2000
    return pl.pallas_call(
        _t3_body,
        grid=(N // BN,),
        in_specs=[
            pl.BlockSpec((NC, BN, H), lambda i: (0, i, 0)),
            pl.BlockSpec((NC, BN, X), lambda i: (0, i, 0)),
            pl.BlockSpec((NC, BN, H), lambda i: (0, i, 0)),
            pl.BlockSpec((NC, BN, X), lambda i: (0, i, 0)),
            pl.BlockSpec((BN, D), lambda i: (i, 0)),
            pl.BlockSpec((BN, 3), lambda i: (i, 0)),
            pl.BlockSpec((D, H), lambda i: (0, 0)),
            pl.BlockSpec((H, H), lambda i: (0, 0)),
            pl.BlockSpec((1, H), lambda i: (0, 0)),
            pl.BlockSpec((H, D), lambda i: (0, 0)),
            pl.BlockSpec((1, D), lambda i: (0, 0)),
        ],
        out_specs=[
            pl.BlockSpec((BN, D), lambda i: (i, 0)),
            pl.BlockSpec((BN, 3), lambda i: (i, 0)),
        ],
        out_shape=[
            jax.ShapeDtypeStruct((N, D), jnp.float32),
            jax.ShapeDtypeStruct((N, 3), jnp.float32),
        ],
    )(ACC2a, ACCXa, ACC2b, ACCXb, h, coord, Wn1h, Wn1a, b_n1, W_n2, b_n2)


@jax.jit
def kernel(h, edge_index, coord, edge_attr, W_e1, b_e1, W_e2, b_e2,
           W_n1, b_n1, W_n2, b_n2, W_c1, b_c1, W_c2):
    row = edge_index[0]
    col = edge_index[1]

    W1r = W_e1[:D]
    W1c = W_e1[D:2 * D]
    w1rad = W_e1[2 * D:2 * D + 1]
    W1a = W_e1[2 * D + 1:]

    CX = jnp.pad(coord, ((0, 0), (0, X - 3)))

    P, Q = _t1(h, W1r, W1c, b_e1.reshape(1, H))

    accs = []
    for c in range(CHUNKS):
        sl = slice(c * EC, (c + 1) * EC)
        G, DF = _s1(row[sl], col[sl], P, Q, CX)
        F, FX = _t2(G, DF, edge_attr[sl], W1a, w1rad, W_e2,
                    b_e2.reshape(1, H), W_c1, b_c1.reshape(1, H), W_c2)
        accs += list(_s2(row[sl], F, FX))

    h_out, coord_out = _t3(accs[0], accs[1], accs[2], accs[3], h, coord,
                           W_n1[:D], W_n1[D:], b_n1.reshape(1, H), W_n2,
                           b_n2.reshape(1, D))
    return (h_out, coord_out, edge_attr)


# speedup vs baseline: 1.3824x; 1.0413x over previous
import functools

import jax
import jax.numpy as jnp
from jax import lax
from jax.experimental import pallas as pl
from jax.experimental.pallas import tpu as pltpu
from jax.experimental.pallas import tpu_sc as plsc

N = 10000
E = 320000
D = 128
H = 128
DE = 16
X = 16
NC = 2
NS = 16
NW = NC * NS
CHUNKS = 2
EC = E // CHUNKS
EW = EC // NW
WIN = 128
NFULL = EW // WIN
TAIL = EW - NFULL * WIN
NPT = N // NS
ZR = 125

_SC_PARAMS = pltpu.CompilerParams(use_tc_tiling_on_sc=False)


def _t1_body(h_ref, w1r_ref, w1c_ref, be1_ref, p_ref, q_ref):
    h = h_ref[...]
    p_ref[...] = h @ w1r_ref[...] + be1_ref[...]
    q_ref[...] = h @ w1c_ref[...]


def _t1(h, W1r, W1c, b_e1):
    BN = 2000
    return pl.pallas_call(
        _t1_body,
        grid=(N // BN,),
        in_specs=[
            pl.BlockSpec((BN, D), lambda i: (i, 0)),
            pl.BlockSpec((D, H), lambda i: (0, 0)),
            pl.BlockSpec((D, H), lambda i: (0, 0)),
            pl.BlockSpec((1, H), lambda i: (0, 0)),
        ],
        out_specs=[
            pl.BlockSpec((BN, H), lambda i: (i, 0)),
            pl.BlockSpec((BN, H), lambda i: (i, 0)),
        ],
        out_shape=[
            jax.ShapeDtypeStruct((N, H), jnp.float32),
            jax.ShapeDtypeStruct((N, H), jnp.float32),
        ],
    )(h, W1r, W1c, b_e1)


_NSLOT = 3


def _s1_body(row_hbm, col_hbm, p_hbm, q_hbm, cx_hbm, g_hbm, df_hbm, *refs):
    cid = lax.axis_index("c")
    sid = lax.axis_index("s")
    wbase = (sid * NC + cid) * EW

    slots = [refs[9 * j:9 * (j + 1)] for j in range(_NSLOT)]

    def start_idx(base, sl):
        ir, ic = sl[0], sl[1]
        si = sl[6]
        return (pltpu.async_copy(row_hbm.at[pl.ds(base, WIN)], ir, si),
                pltpu.async_copy(col_hbm.at[pl.ds(base, WIN)], ic, si))

    def start_gather(sl):
        ir, ic, gr, gc, cr, cc, _, sg, _ = sl
        return (pltpu.async_copy(p_hbm.at[ir], gr, sg),
                pltpu.async_copy(q_hbm.at[ic], gc, sg),
                pltpu.async_copy(cx_hbm.at[ir], cr, sg),
                pltpu.async_copy(cx_hbm.at[ic], cc, sg))

    def compute(sl):
        gr, gc, cr, cc = sl[2], sl[3], sl[4], sl[5]

        @plsc.parallel_loop(0, WIN, unroll=8)
        def _(r):
            for ch in range(H // 16):
                s = pl.ds(ch * 16, 16)
                gr[r, s] = gr[r, s] + gc[r, s]
            cr[r, :] = cr[r, :] - cc[r, :]

    def start_store(base, sl):
        gr, cr, ss = sl[2], sl[4], sl[8]
        return (pltpu.async_copy(gr, g_hbm.at[pl.ds(base, WIN)], ss),
                pltpu.async_copy(cr, df_hbm.at[pl.ds(base, WIN)], ss))

    def body(k, _):
        base = wbase + k * (_NSLOT * WIN)
        ha = [start_idx(base + j * WIN, slots[j]) for j in range(_NSLOT)]
        hg = []
        for j in range(_NSLOT):
            for hh in ha[j]:
                hh.wait()
            hg.append(start_gather(slots[j]))
        hs = []
        for j in range(_NSLOT):
            for hh in hg[j]:
                hh.wait()
            compute(slots[j])
            hs.append(start_store(base + j * WIN, slots[j]))
        for hj in hs:
            for hh in hj:
                hh.wait()
        return ()

    lax.fori_loop(0, NFULL // _NSLOT, body, ())

    ir, ic, gr, gc, cr, cc, si, sg, ss = slots[0]
    base = wbase + NFULL * WIN
    pltpu.sync_copy(row_hbm.at[pl.ds(base, TAIL)], ir.at[pl.ds(0, TAIL)])
    pltpu.sync_copy(col_hbm.at[pl.ds(base, TAIL)], ic.at[pl.ds(0, TAIL)])
    irs = ir.at[pl.ds(0, TAIL)]
    ics = ic.at[pl.ds(0, TAIL)]
    cp1 = pltpu.async_copy(p_hbm.at[irs], gr.at[pl.ds(0, TAIL)], sg)
    cp2 = pltpu.async_copy(q_hbm.at[ics], gc.at[pl.ds(0, TAIL)], sg)
    cp3 = pltpu.async_copy(cx_hbm.at[irs], cr.at[pl.ds(0, TAIL)], sg)
    cp4 = pltpu.async_copy(cx_hbm.at[ics], cc.at[pl.ds(0, TAIL)], sg)
    cp1.wait()
    cp2.wait()
    cp3.wait()
    cp4.wait()

    @plsc.parallel_loop(0, TAIL, unroll=8)
    def _(r):
        for ch in range(H // 16):
            s = pl.ds(ch * 16, 16)
            gr[r, s] = gr[r, s] + gc[r, s]
        cr[r, :] = cr[r, :] - cc[r, :]

    pltpu.sync_copy(gr.at[pl.ds(0, TAIL)], g_hbm.at[pl.ds(base, TAIL)])
    pltpu.sync_copy(cr.at[pl.ds(0, TAIL)], df_hbm.at[pl.ds(base, TAIL)])


def _s1(row, col, P, Q, CX):
    mesh = plsc.VectorSubcoreMesh(core_axis_name="c", subcore_axis_name="s")
    slot_scratch = []
    for _ in range(_NSLOT):
        slot_scratch += [
            pltpu.VMEM((WIN,), jnp.int32),
            pltpu.VMEM((WIN,), jnp.int32),
            pltpu.VMEM((WIN, H), jnp.float32),
            pltpu.VMEM((WIN, H), jnp.float32),
            pltpu.VMEM((WIN, X), jnp.float32),
            pltpu.VMEM((WIN, X), jnp.float32),
            pltpu.SemaphoreType.DMA,
            pltpu.SemaphoreType.DMA,
            pltpu.SemaphoreType.DMA,
        ]
    return pl.kernel(
        _s1_body,
        out_type=[
            jax.ShapeDtypeStruct((EC, H), jnp.float32),
            jax.ShapeDtypeStruct((EC, X), jnp.float32),
        ],
        name="s1_gather",
        mesh=mesh,
        compiler_params=_SC_PARAMS,
        scratch_types=slot_scratch,
    )(row, col, P, Q, CX)


def _t2_body(g_ref, df_ref, ea_ref, w1a_ref, w1rad_ref, we2_ref,
             be2_ref, wc1_ref, bc1_ref, wc2_ref, f_ref, fx_ref):
    diff = df_ref[:, :3]
    radial = jnp.sum(diff * diff, axis=1, keepdims=True)
    m_in = g_ref[...] + radial * w1rad_ref[...] + ea_ref[...] @ w1a_ref[...]
    m = jax.nn.silu(m_in)
    ef = jax.nn.silu(m @ we2_ref[...] + be2_ref[...])
    c1 = jax.nn.silu(ef @ wc1_ref[...] + bc1_ref[...])
    phi = c1 @ wc2_ref[...]
    bt = ef.shape[0]
    f_ref[...] = ef
    fx_ref[...] = jnp.concatenate(
        [diff * phi, jnp.ones((bt, 1), jnp.float32),
         jnp.zeros((bt, X - 4), jnp.float32)], axis=1)


def _t2(G, DF, edge_attr, W1a, w1rad, W_e2, b_e2, W_c1, b_c1, W_c2):
    BT = 1280
    return pl.pallas_call(
        _t2_body,
        grid=(EC // BT,),
        in_specs=[
            pl.BlockSpec((BT, H), lambda i: (i, 0)),
            pl.BlockSpec((BT, X), lambda i: (i, 0)),
            pl.BlockSpec((BT, DE), lambda i: (i, 0)),
            pl.BlockSpec((DE, H), lambda i: (0, 0)),
            pl.BlockSpec((1, H), lambda i: (0, 0)),
            pl.BlockSpec((H, H), lambda i: (0, 0)),
            pl.BlockSpec((1, H), lambda i: (0, 0)),
            pl.BlockSpec((H, H), lambda i: (0, 0)),
            pl.BlockSpec((1, H), lambda i: (0, 0)),
            pl.BlockSpec((H, 1), lambda i: (0, 0)),
        ],
        out_specs=[
            pl.BlockSpec((BT, H), lambda i: (i, 0)),
            pl.BlockSpec((BT, X), lambda i: (i, 0)),
        ],
        out_shape=[
            jax.ShapeDtypeStruct((EC, H), jnp.float32),
            jax.ShapeDtypeStruct((EC, X), jnp.float32),
        ],
    )(G, DF, edge_attr, W1a, w1rad, W_e2, b_e2, W_c1, b_c1, W_c2)


def _s2_body(row_hbm, f_hbm, fx_hbm, acc2_hbm, accx2_hbm,
             ib, fb, fxb, ib1, fb1, fxb1, acc, accx, sl0, sl1):
    cid = lax.axis_index("c")
    sid = lax.axis_index("s")
    wbase = (sid * NC + cid) * EW

    zero16 = jnp.zeros((16,), jnp.float32)

    def zrow(r, _):
        for ch in range(H // 16):
            fb[r, pl.ds(ch * 16, 16)] = zero16
        fxb[r, :] = zero16
        return ()

    lax.fori_loop(0, WIN, zrow, ())

    def zchunk(k, _):
        pltpu.sync_copy(fb.at[pl.ds(0, ZR)],
                        acc.at[pl.ds(sid * NPT + k * ZR, ZR)])
        pltpu.sync_copy(fxb.at[pl.ds(0, ZR)],
                        accx.at[pl.ds(sid * NPT + k * ZR, ZR)])
        return ()

    lax.fori_loop(0, NPT // ZR, zchunk, ())
    plsc.subcore_barrier()

    def start_loads(base, ib_, fb_, fxb_, sem):
        return (pltpu.async_copy(row_hbm.at[pl.ds(base, WIN)], ib_, sem),
                pltpu.async_copy(f_hbm.at[pl.ds(base, WIN)], fb_, sem),
                pltpu.async_copy(fx_hbm.at[pl.ds(base, WIN)], fxb_, sem))

    def scatter(ib_, fb_, fxb_):
        pltpu.sync_copy(fb_, acc.at[ib_], add=True)
        pltpu.sync_copy(fxb_, accx.at[ib_], add=True)

    def body(k, _):
        b0 = wbase + 2 * k * WIN
        h0 = start_loads(b0, ib, fb, fxb, sl0)
        h1 = start_loads(b0 + WIN, ib1, fb1, fxb1, sl1)
        for hh in h0:
            hh.wait()
        scatter(ib, fb, fxb)
        for hh in h1:
            hh.wait()
        scatter(ib1, fb1, fxb1)
        return ()

    lax.fori_loop(0, NFULL // 2, body, ())

    lastb = wbase + (NFULL - 1) * WIN
    h0 = start_loads(lastb, ib, fb, fxb, sl0)
    for hh in h0:
        hh.wait()
    scatter(ib, fb, fxb)

    base = wbase + NFULL * WIN
    pltpu.sync_copy(row_hbm.at[pl.ds(base, TAIL)], ib1.at[pl.ds(0, TAIL)])
    ibs = ib1.at[pl.ds(0, TAIL)]
    pltpu.sync_copy(f_hbm.at[pl.ds(base, TAIL)], fb1.at[pl.ds(0, TAIL)])
    pltpu.sync_copy(fx_hbm.at[pl.ds(base, TAIL)], fxb1.at[pl.ds(0, TAIL)])
    pltpu.sync_copy(fb1.at[pl.ds(0, TAIL)], acc.at[ibs], add=True)
    pltpu.sync_copy(fxb1.at[pl.ds(0, TAIL)], accx.at[ibs], add=True)

    plsc.subcore_barrier()

    def flush(k, _):
        off = sid * NPT + k * ZR
        pltpu.sync_copy(acc.at[pl.ds(off, ZR)], acc2_hbm.at[cid, pl.ds(off, ZR)])
        pltpu.sync_copy(accx.at[pl.ds(off, ZR)],
                        accx2_hbm.at[cid, pl.ds(off, ZR)])
        return ()

    lax.fori_loop(0, NPT // ZR, flush, ())


def _s2(row, F, FX):
    mesh = plsc.VectorSubcoreMesh(core_axis_name="c", subcore_axis_name="s")
    return pl.kernel(
        _s2_body,
        out_type=[
            jax.ShapeDtypeStruct((NC, N, H), jnp.float32),
            jax.ShapeDtypeStruct((NC, N, X), jnp.float32),
        ],
        name="s2_scatter",
        mesh=mesh,
        compiler_params=_SC_PARAMS,
        scratch_types=[
            pltpu.VMEM((WIN,), jnp.int32),
            pltpu.VMEM((WIN, H), jnp.float32),
            pltpu.VMEM((WIN, X), jnp.float32),
            pltpu.VMEM((WIN,), jnp.int32),
            pltpu.VMEM((WIN, H), jnp.float32),
            pltpu.VMEM((WIN, X), jnp.float32),
            pltpu.VMEM_SHARED((N, H), jnp.float32),
            pltpu.VMEM_SHARED((N, X), jnp.float32),
            pltpu.SemaphoreType.DMA,
            pltpu.SemaphoreType.DMA,
        ],
    )(row, F, FX)


def _t3_body(acc2a_ref, accxa_ref, acc2b_ref, accxb_ref, h_ref, c_ref,
             wn1h_ref, wn1a_ref, bn1_ref, wn2_ref, bn2_ref, ho_ref, co_ref):
    agg_h = (acc2a_ref[0] + acc2a_ref[1]) + (acc2b_ref[0] + acc2b_ref[1])
    accx = (accxa_ref[0] + accxa_ref[1]) + (accxb_ref[0] + accxb_ref[1])
    sums = accx[:, :3]
    cnt = accx[:, 3:4]
    h = h_ref[...]
    t = jax.nn.silu(h @ wn1h_ref[...] + agg_h @ wn1a_ref[...] + bn1_ref[...])
    ho_ref[...] = h + t @ wn2_ref[...] + bn2_ref[...]
    co_ref[...] = c_ref[...] + sums / jnp.maximum(cnt, 1.0)


def _t3(ACC2a, ACCXa, ACC2b, ACCXb, h, coord, Wn1h, Wn1a, b_n1, W_n2, b_n2):
    BN = 2000
    return pl.pallas_call(
        _t3_body,
        grid=(N // BN,),
        in_specs=[
            pl.BlockSpec((NC, BN, H), lambda i: (0, i, 0)),
            pl.BlockSpec((NC, BN, X), lambda i: (0, i, 0)),
            pl.BlockSpec((NC, BN, H), lambda i: (0, i, 0)),
            pl.BlockSpec((NC, BN, X), lambda i: (0, i, 0)),
            pl.BlockSpec((BN, D), lambda i: (i, 0)),
            pl.BlockSpec((BN, 3), lambda i: (i, 0)),
            pl.BlockSpec((D, H), lambda i: (0, 0)),
            pl.BlockSpec((H, H), lambda i: (0, 0)),
            pl.BlockSpec((1, H), lambda i: (0, 0)),
            pl.BlockSpec((H, D), lambda i: (0, 0)),
            pl.BlockSpec((1, D), lambda i: (0, 0)),
        ],
        out_specs=[
            pl.BlockSpec((BN, D), lambda i: (i, 0)),
            pl.BlockSpec((BN, 3), lambda i: (i, 0)),
        ],
        out_shape=[
            jax.ShapeDtypeStruct((N, D), jnp.float32),
            jax.ShapeDtypeStruct((N, 3), jnp.float32),
        ],
    )(ACC2a, ACCXa, ACC2b, ACCXb, h, coord, Wn1h, Wn1a, b_n1, W_n2, b_n2)


@jax.jit
def kernel(h, edge_index, coord, edge_attr, W_e1, b_e1, W_e2, b_e2,
           W_n1, b_n1, W_n2, b_n2, W_c1, b_c1, W_c2):
    row = edge_index[0]
    col = edge_index[1]

    W1r = W_e1[:D]
    W1c = W_e1[D:2 * D]
    w1rad = W_e1[2 * D:2 * D + 1]
    W1a = W_e1[2 * D + 1:]

    CX = jnp.pad(coord, ((0, 0), (0, X - 3)))

    P, Q = _t1(h, W1r, W1c, b_e1.reshape(1, H))

    accs = []
    for c in range(CHUNKS):
        sl = slice(c * EC, (c + 1) * EC)
        G, DF = _s1(row[sl], col[sl], P, Q, CX)
        F, FX = _t2(G, DF, edge_attr[sl], W1a, w1rad, W_e2,
                    b_e2.reshape(1, H), W_c1, b_c1.reshape(1, H), W_c2)
        accs += list(_s2(row[sl], F, FX))

    h_out, coord_out = _t3(accs[0], accs[1], accs[2], accs[3], h, coord,
                           W_n1[:D], W_n1[D:], b_n1.reshape(1, H), W_n2,
                           b_n2.reshape(1, D))
    return (h_out, coord_out, edge_attr)


# speedup vs baseline: 1.4420x; 1.0431x over previous
import functools

import jax
import jax.numpy as jnp
from jax import lax
from jax.experimental import pallas as pl
from jax.experimental.pallas import tpu as pltpu
from jax.experimental.pallas import tpu_sc as plsc

N = 10000
E = 320000
D = 128
H = 128
DE = 16
X = 16
NC = 2
NS = 16
NW = NC * NS
CHUNKS = 5
EC = E // CHUNKS
EW = EC // NW
WIN = 128
NFULL = EW // WIN
TAIL = EW - NFULL * WIN
NPT = N // NS
ZR = 125

_SC_PARAMS = pltpu.CompilerParams(use_tc_tiling_on_sc=False)


def _t1_body(h_ref, w1r_ref, w1c_ref, be1_ref, p_ref, q_ref):
    h = h_ref[...]
    p_ref[...] = h @ w1r_ref[...] + be1_ref[...]
    q_ref[...] = h @ w1c_ref[...]


def _t1(h, W1r, W1c, b_e1):
    BN = 2000
    return pl.pallas_call(
        _t1_body,
        grid=(N // BN,),
        in_specs=[
            pl.BlockSpec((BN, D), lambda i: (i, 0)),
            pl.BlockSpec((D, H), lambda i: (0, 0)),
            pl.BlockSpec((D, H), lambda i: (0, 0)),
            pl.BlockSpec((1, H), lambda i: (0, 0)),
        ],
        out_specs=[
            pl.BlockSpec((BN, H), lambda i: (i, 0)),
            pl.BlockSpec((BN, H), lambda i: (i, 0)),
        ],
        out_shape=[
            jax.ShapeDtypeStruct((N, H), jnp.float32),
            jax.ShapeDtypeStruct((N, H), jnp.float32),
        ],
    )(h, W1r, W1c, b_e1)


_NSLOT = 3


def _s1_body(row_hbm, col_hbm, p_hbm, q_hbm, cx_hbm, g_hbm, df_hbm, *refs):
    cid = lax.axis_index("c")
    sid = lax.axis_index("s")
    wbase = (sid * NC + cid) * EW

    slots = [refs[9 * j:9 * (j + 1)] for j in range(_NSLOT)]

    def start_idx(base, sl):
        ir, ic = sl[0], sl[1]
        si = sl[6]
        return (pltpu.async_copy(row_hbm.at[pl.ds(base, WIN)], ir, si),
                pltpu.async_copy(col_hbm.at[pl.ds(base, WIN)], ic, si))

    def start_gather(sl):
        ir, ic, gr, gc, cr, cc, _, sg, _ = sl
        return (pltpu.async_copy(p_hbm.at[ir], gr, sg),
                pltpu.async_copy(q_hbm.at[ic], gc, sg),
                pltpu.async_copy(cx_hbm.at[ir], cr, sg),
                pltpu.async_copy(cx_hbm.at[ic], cc, sg))

    def compute(sl):
        gr, gc, cr, cc = sl[2], sl[3], sl[4], sl[5]

        @plsc.parallel_loop(0, WIN, unroll=8)
        def _(r):
            for ch in range(H // 16):
                s = pl.ds(ch * 16, 16)
                gr[r, s] = gr[r, s] + gc[r, s]
            cr[r, :] = cr[r, :] - cc[r, :]

    def start_store(base, sl):
        gr, cr, ss = sl[2], sl[4], sl[8]
        return (pltpu.async_copy(gr, g_hbm.at[pl.ds(base, WIN)], ss),
                pltpu.async_copy(cr, df_hbm.at[pl.ds(base, WIN)], ss))

    def body(k, _):
        base = wbase + k * (_NSLOT * WIN)
        ha = [start_idx(base + j * WIN, slots[j]) for j in range(_NSLOT)]
        hg = []
        for j in range(_NSLOT):
            for hh in ha[j]:
                hh.wait()
            hg.append(start_gather(slots[j]))
        hs = []
        for j in range(_NSLOT):
            for hh in hg[j]:
                hh.wait()
            compute(slots[j])
            hs.append(start_store(base + j * WIN, slots[j]))
        for hj in hs:
            for hh in hj:
                hh.wait()
        return ()

    lax.fori_loop(0, NFULL // _NSLOT, body, ())

    ir, ic, gr, gc, cr, cc, si, sg, ss = slots[0]
    base = wbase + NFULL * WIN
    pltpu.sync_copy(row_hbm.at[pl.ds(base, TAIL)], ir.at[pl.ds(0, TAIL)])
    pltpu.sync_copy(col_hbm.at[pl.ds(base, TAIL)], ic.at[pl.ds(0, TAIL)])
    irs = ir.at[pl.ds(0, TAIL)]
    ics = ic.at[pl.ds(0, TAIL)]
    cp1 = pltpu.async_copy(p_hbm.at[irs], gr.at[pl.ds(0, TAIL)], sg)
    cp2 = pltpu.async_copy(q_hbm.at[ics], gc.at[pl.ds(0, TAIL)], sg)
    cp3 = pltpu.async_copy(cx_hbm.at[irs], cr.at[pl.ds(0, TAIL)], sg)
    cp4 = pltpu.async_copy(cx_hbm.at[ics], cc.at[pl.ds(0, TAIL)], sg)
    cp1.wait()
    cp2.wait()
    cp3.wait()
    cp4.wait()

    @plsc.parallel_loop(0, TAIL, unroll=8)
    def _(r):
        for ch in range(H // 16):
            s = pl.ds(ch * 16, 16)
            gr[r, s] = gr[r, s] + gc[r, s]
        cr[r, :] = cr[r, :] - cc[r, :]

    pltpu.sync_copy(gr.at[pl.ds(0, TAIL)], g_hbm.at[pl.ds(base, TAIL)])
    pltpu.sync_copy(cr.at[pl.ds(0, TAIL)], df_hbm.at[pl.ds(base, TAIL)])


def _s1(row, col, P, Q, CX):
    mesh = plsc.VectorSubcoreMesh(core_axis_name="c", subcore_axis_name="s")
    slot_scratch = []
    for _ in range(_NSLOT):
        slot_scratch += [
            pltpu.VMEM((WIN,), jnp.int32),
            pltpu.VMEM((WIN,), jnp.int32),
            pltpu.VMEM((WIN, H), jnp.float32),
            pltpu.VMEM((WIN, H), jnp.float32),
            pltpu.VMEM((WIN, X), jnp.float32),
            pltpu.VMEM((WIN, X), jnp.float32),
            pltpu.SemaphoreType.DMA,
            pltpu.SemaphoreType.DMA,
            pltpu.SemaphoreType.DMA,
        ]
    return pl.kernel(
        _s1_body,
        out_type=[
            jax.ShapeDtypeStruct((EC, H), jnp.float32),
            jax.ShapeDtypeStruct((EC, X), jnp.float32),
        ],
        name="s1_gather",
        mesh=mesh,
        compiler_params=_SC_PARAMS,
        scratch_types=slot_scratch,
    )(row, col, P, Q, CX)


def _t2_body(g_ref, df_ref, ea_ref, w1a_ref, w1rad_ref, we2_ref,
             be2_ref, wc1_ref, bc1_ref, wc2_ref, f_ref, fx_ref):
    diff = df_ref[:, :3]
    radial = jnp.sum(diff * diff, axis=1, keepdims=True)
    m_in = g_ref[...] + radial * w1rad_ref[...] + ea_ref[...] @ w1a_ref[...]
    m = jax.nn.silu(m_in)
    ef = jax.nn.silu(m @ we2_ref[...] + be2_ref[...])
    c1 = jax.nn.silu(ef @ wc1_ref[...] + bc1_ref[...])
    phi = c1 @ wc2_ref[...]
    bt = ef.shape[0]
    f_ref[...] = ef
    fx_ref[...] = jnp.concatenate(
        [diff * phi, jnp.ones((bt, 1), jnp.float32),
         jnp.zeros((bt, X - 4), jnp.float32)], axis=1)


def _t2(G, DF, edge_attr, W1a, w1rad, W_e2, b_e2, W_c1, b_c1, W_c2):
    BT = 1280
    return pl.pallas_call(
        _t2_body,
        grid=(EC // BT,),
        in_specs=[
            pl.BlockSpec((BT, H), lambda i: (i, 0)),
            pl.BlockSpec((BT, X), lambda i: (i, 0)),
            pl.BlockSpec((BT, DE), lambda i: (i, 0)),
            pl.BlockSpec((DE, H), lambda i: (0, 0)),
            pl.BlockSpec((1, H), lambda i: (0, 0)),
            pl.BlockSpec((H, H), lambda i: (0, 0)),
            pl.BlockSpec((1, H), lambda i: (0, 0)),
            pl.BlockSpec((H, H), lambda i: (0, 0)),
            pl.BlockSpec((1, H), lambda i: (0, 0)),
            pl.BlockSpec((H, 1), lambda i: (0, 0)),
        ],
        out_specs=[
            pl.BlockSpec((BT, H), lambda i: (i, 0)),
            pl.BlockSpec((BT, X), lambda i: (i, 0)),
        ],
        out_shape=[
            jax.ShapeDtypeStruct((EC, H), jnp.float32),
            jax.ShapeDtypeStruct((EC, X), jnp.float32),
        ],
    )(G, DF, edge_attr, W1a, w1rad, W_e2, b_e2, W_c1, b_c1, W_c2)


def _s2_body(row_hbm, f_hbm, fx_hbm, acc2_hbm, accx2_hbm,
             ib, fb, fxb, ib1, fb1, fxb1, acc, accx, sl0, sl1):
    cid = lax.axis_index("c")
    sid = lax.axis_index("s")
    wbase = (sid * NC + cid) * EW

    zero16 = jnp.zeros((16,), jnp.float32)

    def zrow(r, _):
        for ch in range(H // 16):
            fb[r, pl.ds(ch * 16, 16)] = zero16
        fxb[r, :] = zero16
        return ()

    lax.fori_loop(0, WIN, zrow, ())

    def zchunk(k, _):
        pltpu.sync_copy(fb.at[pl.ds(0, ZR)],
                        acc.at[pl.ds(sid * NPT + k * ZR, ZR)])
        pltpu.sync_copy(fxb.at[pl.ds(0, ZR)],
                        accx.at[pl.ds(sid * NPT + k * ZR, ZR)])
        return ()

    lax.fori_loop(0, NPT // ZR, zchunk, ())
    plsc.subcore_barrier()

    def start_loads(base, ib_, fb_, fxb_, sem):
        return (pltpu.async_copy(row_hbm.at[pl.ds(base, WIN)], ib_, sem),
                pltpu.async_copy(f_hbm.at[pl.ds(base, WIN)], fb_, sem),
                pltpu.async_copy(fx_hbm.at[pl.ds(base, WIN)], fxb_, sem))

    def scatter(ib_, fb_, fxb_):
        pltpu.sync_copy(fb_, acc.at[ib_], add=True)
        pltpu.sync_copy(fxb_, accx.at[ib_], add=True)

    def body(k, _):
        b0 = wbase + 2 * k * WIN
        h0 = start_loads(b0, ib, fb, fxb, sl0)
        h1 = start_loads(b0 + WIN, ib1, fb1, fxb1, sl1)
        for hh in h0:
            hh.wait()
        scatter(ib, fb, fxb)
        for hh in h1:
            hh.wait()
        scatter(ib1, fb1, fxb1)
        return ()

    lax.fori_loop(0, NFULL // 2, body, ())

    lastb = wbase + (NFULL - 1) * WIN
    h0 = start_loads(lastb, ib, fb, fxb, sl0)
    for hh in h0:
        hh.wait()
    scatter(ib, fb, fxb)

    base = wbase + NFULL * WIN
    pltpu.sync_copy(row_hbm.at[pl.ds(base, TAIL)], ib1.at[pl.ds(0, TAIL)])
    ibs = ib1.at[pl.ds(0, TAIL)]
    pltpu.sync_copy(f_hbm.at[pl.ds(base, TAIL)], fb1.at[pl.ds(0, TAIL)])
    pltpu.sync_copy(fx_hbm.at[pl.ds(base, TAIL)], fxb1.at[pl.ds(0, TAIL)])
    pltpu.sync_copy(fb1.at[pl.ds(0, TAIL)], acc.at[ibs], add=True)
    pltpu.sync_copy(fxb1.at[pl.ds(0, TAIL)], accx.at[ibs], add=True)

    plsc.subcore_barrier()

    def flush(k, _):
        off = sid * NPT + k * ZR
        pltpu.sync_copy(acc.at[pl.ds(off, ZR)], acc2_hbm.at[cid, pl.ds(off, ZR)])
        pltpu.sync_copy(accx.at[pl.ds(off, ZR)],
                        accx2_hbm.at[cid, pl.ds(off, ZR)])
        return ()

    lax.fori_loop(0, NPT // ZR, flush, ())


def _s2(row, F, FX):
    mesh = plsc.VectorSubcoreMesh(core_axis_name="c", subcore_axis_name="s")
    return pl.kernel(
        _s2_body,
        out_type=[
            jax.ShapeDtypeStruct((NC, N, H), jnp.float32),
            jax.ShapeDtypeStruct((NC, N, X), jnp.float32),
        ],
        name="s2_scatter",
        mesh=mesh,
        compiler_params=_SC_PARAMS,
        scratch_types=[
            pltpu.VMEM((WIN,), jnp.int32),
            pltpu.VMEM((WIN, H), jnp.float32),
            pltpu.VMEM((WIN, X), jnp.float32),
            pltpu.VMEM((WIN,), jnp.int32),
            pltpu.VMEM((WIN, H), jnp.float32),
            pltpu.VMEM((WIN, X), jnp.float32),
            pltpu.VMEM_SHARED((N, H), jnp.float32),
            pltpu.VMEM_SHARED((N, X), jnp.float32),
            pltpu.SemaphoreType.DMA,
            pltpu.SemaphoreType.DMA,
        ],
    )(row, F, FX)


def _t3_body(*refs):
    acc_refs = refs[:2 * CHUNKS]
    (h_ref, c_ref, wn1h_ref, wn1a_ref, bn1_ref, wn2_ref, bn2_ref,
     ho_ref, co_ref) = refs[2 * CHUNKS:]
    agg_h = sum(r[0] + r[1] for r in acc_refs[0::2])
    accx = sum(r[0] + r[1] for r in acc_refs[1::2])
    sums = accx[:, :3]
    cnt = accx[:, 3:4]
    h = h_ref[...]
    t = jax.nn.silu(h @ wn1h_ref[...] + agg_h @ wn1a_ref[...] + bn1_ref[...])
    ho_ref[...] = h + t @ wn2_ref[...] + bn2_ref[...]
    co_ref[...] = c_ref[...] + sums / jnp.maximum(cnt, 1.0)


def _t3(accs, h, coord, Wn1h, Wn1a, b_n1, W_n2, b_n2):
    BN = 2000
    acc_specs = []
    for _ in range(CHUNKS):
        acc_specs += [
            pl.BlockSpec((NC, BN, H), lambda i: (0, i, 0)),
            pl.BlockSpec((NC, BN, X), lambda i: (0, i, 0)),
        ]
    return pl.pallas_call(
        _t3_body,
        grid=(N // BN,),
        in_specs=acc_specs + [
            pl.BlockSpec((BN, D), lambda i: (i, 0)),
            pl.BlockSpec((BN, 3), lambda i: (i, 0)),
            pl.BlockSpec((D, H), lambda i: (0, 0)),
            pl.BlockSpec((H, H), lambda i: (0, 0)),
            pl.BlockSpec((1, H), lambda i: (0, 0)),
            pl.BlockSpec((H, D), lambda i: (0, 0)),
            pl.BlockSpec((1, D), lambda i: (0, 0)),
        ],
        out_specs=[
            pl.BlockSpec((BN, D), lambda i: (i, 0)),
            pl.BlockSpec((BN, 3), lambda i: (i, 0)),
        ],
        out_shape=[
            jax.ShapeDtypeStruct((N, D), jnp.float32),
            jax.ShapeDtypeStruct((N, 3), jnp.float32),
        ],
    )(*accs, h, coord, Wn1h, Wn1a, b_n1, W_n2, b_n2)


@jax.jit
def kernel(h, edge_index, coord, edge_attr, W_e1, b_e1, W_e2, b_e2,
           W_n1, b_n1, W_n2, b_n2, W_c1, b_c1, W_c2):
    row = edge_index[0]
    col = edge_index[1]

    W1r = W_e1[:D]
    W1c = W_e1[D:2 * D]
    w1rad = W_e1[2 * D:2 * D + 1]
    W1a = W_e1[2 * D + 1:]

    CX = jnp.pad(coord, ((0, 0), (0, X - 3)))

    P, Q = _t1(h, W1r, W1c, b_e1.reshape(1, H))

    accs = []
    for c in range(CHUNKS):
        sl = slice(c * EC, (c + 1) * EC)
        G, DF = _s1(row[sl], col[sl], P, Q, CX)
        F, FX = _t2(G, DF, edge_attr[sl], W1a, w1rad, W_e2,
                    b_e2.reshape(1, H), W_c1, b_c1.reshape(1, H), W_c2)
        accs += list(_s2(row[sl], F, FX))

    h_out, coord_out = _t3(accs, h, coord, W_n1[:D], W_n1[D:],
                           b_n1.reshape(1, H), W_n2, b_n2.reshape(1, D))
    return (h_out, coord_out, edge_attr)


# speedup vs baseline: 1.4697x; 1.0192x over previous
import functools

import jax
import jax.numpy as jnp
from jax import lax
from jax.experimental import pallas as pl
from jax.experimental.pallas import tpu as pltpu
from jax.experimental.pallas import tpu_sc as plsc

N = 10000
E = 320000
D = 128
H = 128
DE = 16
X = 16
NC = 2
NS = 16
NW = NC * NS
CHUNKS = 5
EC = E // CHUNKS
EW = EC // NW
WIN = 128
NFULL = EW // WIN
TAIL = EW - NFULL * WIN
NPT = N // NS
ZR = 125

_SC_PARAMS = pltpu.CompilerParams(use_tc_tiling_on_sc=False)
_SC_PARAMS_S1 = pltpu.CompilerParams(use_tc_tiling_on_sc=False,
                                     needs_layout_passes=False)


def _t1_body(h_ref, w1r_ref, w1c_ref, be1_ref, p_ref, q_ref):
    h = h_ref[...]
    p_ref[...] = (h @ w1r_ref[...] + be1_ref[...]).astype(jnp.bfloat16)
    q_ref[...] = (h @ w1c_ref[...]).astype(jnp.bfloat16)


def _t1(h, W1r, W1c, b_e1):
    BN = 2000
    return pl.pallas_call(
        _t1_body,
        grid=(N // BN,),
        in_specs=[
            pl.BlockSpec((BN, D), lambda i: (i, 0)),
            pl.BlockSpec((D, H), lambda i: (0, 0)),
            pl.BlockSpec((D, H), lambda i: (0, 0)),
            pl.BlockSpec((1, H), lambda i: (0, 0)),
        ],
        out_specs=[
            pl.BlockSpec((BN, H), lambda i: (i, 0)),
            pl.BlockSpec((BN, H), lambda i: (i, 0)),
        ],
        out_shape=[
            jax.ShapeDtypeStruct((N, H), jnp.bfloat16),
            jax.ShapeDtypeStruct((N, H), jnp.bfloat16),
        ],
    )(h, W1r, W1c, b_e1)


_NSLOT = 3


def _s1_body(row_hbm, col_hbm, p_hbm, q_hbm, cx_hbm, g_hbm, df_hbm, *refs):
    cid = lax.axis_index("c")
    sid = lax.axis_index("s")
    wbase = (sid * NC + cid) * EW

    slots = [refs[10 * j:10 * (j + 1)] for j in range(_NSLOT)]

    def start_idx(base, sl):
        ir, ic = sl[0], sl[1]
        si = sl[7]
        return (pltpu.async_copy(row_hbm.at[pl.ds(base, WIN)], ir, si),
                pltpu.async_copy(col_hbm.at[pl.ds(base, WIN)], ic, si))

    def start_gather(sl):
        ir, ic, gr, gc, go, cr, cc, _, sg, _ = sl
        return (pltpu.async_copy(p_hbm.at[ir], gr, sg),
                pltpu.async_copy(q_hbm.at[ic], gc, sg),
                pltpu.async_copy(cx_hbm.at[ir], cr, sg),
                pltpu.async_copy(cx_hbm.at[ic], cc, sg))

    def compute(sl):
        gr, gc, go, cr, cc = sl[2], sl[3], sl[4], sl[5], sl[6]

        @plsc.parallel_loop(0, WIN, unroll=8)
        def _(r):
            for grp in range(H // 32):
                s = pl.ds(grp * 32, 32)
                vs = gr[r, s] + gc[r, s]
                lo, hi = plsc.unpack(vs, format=plsc.PackFormat.INTERLEAVED)
                go[r, pl.ds(grp * 32, 16)] = lo
                go[r, pl.ds(grp * 32 + 16, 16)] = hi
            cr[r, :] = cr[r, :] - cc[r, :]

    def start_store(base, sl):
        go, cr, ss = sl[4], sl[5], sl[9]
        return (pltpu.async_copy(go, g_hbm.at[pl.ds(base, WIN)], ss),
                pltpu.async_copy(cr, df_hbm.at[pl.ds(base, WIN)], ss))

    def body(k, _):
        base = wbase + k * (_NSLOT * WIN)
        ha = [start_idx(base + j * WIN, slots[j]) for j in range(_NSLOT)]
        hg = []
        for j in range(_NSLOT):
            for hh in ha[j]:
                hh.wait()
            hg.append(start_gather(slots[j]))
        hs = []
        for j in range(_NSLOT):
            for hh in hg[j]:
                hh.wait()
            compute(slots[j])
            hs.append(start_store(base + j * WIN, slots[j]))
        for hj in hs:
            for hh in hj:
                hh.wait()
        return ()

    lax.fori_loop(0, NFULL // _NSLOT, body, ())

    ir, ic, gr, gc, go, cr, cc, si, sg, ss = slots[0]
    base = wbase + NFULL * WIN
    pltpu.sync_copy(row_hbm.at[pl.ds(base, TAIL)], ir.at[pl.ds(0, TAIL)])
    pltpu.sync_copy(col_hbm.at[pl.ds(base, TAIL)], ic.at[pl.ds(0, TAIL)])
    irs = ir.at[pl.ds(0, TAIL)]
    ics = ic.at[pl.ds(0, TAIL)]
    cp1 = pltpu.async_copy(p_hbm.at[irs], gr.at[pl.ds(0, TAIL)], sg)
    cp2 = pltpu.async_copy(q_hbm.at[ics], gc.at[pl.ds(0, TAIL)], sg)
    cp3 = pltpu.async_copy(cx_hbm.at[irs], cr.at[pl.ds(0, TAIL)], sg)
    cp4 = pltpu.async_copy(cx_hbm.at[ics], cc.at[pl.ds(0, TAIL)], sg)
    cp1.wait()
    cp2.wait()
    cp3.wait()
    cp4.wait()

    @plsc.parallel_loop(0, TAIL, unroll=8)
    def _(r):
        for grp in range(H // 32):
            s = pl.ds(grp * 32, 32)
            vs = gr[r, s] + gc[r, s]
            lo, hi = plsc.unpack(vs, format=plsc.PackFormat.INTERLEAVED)
            go[r, pl.ds(grp * 32, 16)] = lo
            go[r, pl.ds(grp * 32 + 16, 16)] = hi
        cr[r, :] = cr[r, :] - cc[r, :]

    pltpu.sync_copy(go.at[pl.ds(0, TAIL)], g_hbm.at[pl.ds(base, TAIL)])
    pltpu.sync_copy(cr.at[pl.ds(0, TAIL)], df_hbm.at[pl.ds(base, TAIL)])


def _s1(row, col, P, Q, CX):
    mesh = plsc.VectorSubcoreMesh(core_axis_name="c", subcore_axis_name="s")
    slot_scratch = []
    for _ in range(_NSLOT):
        slot_scratch += [
            pltpu.VMEM((WIN,), jnp.int32),
            pltpu.VMEM((WIN,), jnp.int32),
            pltpu.VMEM((WIN, H), jnp.bfloat16),
            pltpu.VMEM((WIN, H), jnp.bfloat16),
            pltpu.VMEM((WIN, H), jnp.float32),
            pltpu.VMEM((WIN, X), jnp.float32),
            pltpu.VMEM((WIN, X), jnp.float32),
            pltpu.SemaphoreType.DMA,
            pltpu.SemaphoreType.DMA,
            pltpu.SemaphoreType.DMA,
        ]
    return pl.kernel(
        _s1_body,
        out_type=[
            jax.ShapeDtypeStruct((EC, H), jnp.float32),
            jax.ShapeDtypeStruct((EC, X), jnp.float32),
        ],
        name="s1_gather",
        mesh=mesh,
        compiler_params=_SC_PARAMS_S1,
        scratch_types=slot_scratch,
    )(row, col, P, Q, CX)


def _t2_body(g_ref, df_ref, ea_ref, w1a_ref, w1rad_ref, we2_ref,
             be2_ref, wc1_ref, bc1_ref, wc2_ref, f_ref, fx_ref):
    diff = df_ref[:, :3]
    radial = jnp.sum(diff * diff, axis=1, keepdims=True)
    m_in = g_ref[...] + radial * w1rad_ref[...] + ea_ref[...] @ w1a_ref[...]
    m = jax.nn.silu(m_in)
    ef = jax.nn.silu(m @ we2_ref[...] + be2_ref[...])
    c1 = jax.nn.silu(ef @ wc1_ref[...] + bc1_ref[...])
    phi = c1 @ wc2_ref[...]
    bt = ef.shape[0]
    f_ref[...] = ef
    fx_ref[...] = jnp.concatenate(
        [diff * phi, jnp.ones((bt, 1), jnp.float32),
         jnp.zeros((bt, X - 4), jnp.float32)], axis=1)


def _t2(G, DF, edge_attr, W1a, w1rad, W_e2, b_e2, W_c1, b_c1, W_c2):
    BT = 1280
    return pl.pallas_call(
        _t2_body,
        grid=(EC // BT,),
        in_specs=[
            pl.BlockSpec((BT, H), lambda i: (i, 0)),
            pl.BlockSpec((BT, X), lambda i: (i, 0)),
            pl.BlockSpec((BT, DE), lambda i: (i, 0)),
            pl.BlockSpec((DE, H), lambda i: (0, 0)),
            pl.BlockSpec((1, H), lambda i: (0, 0)),
            pl.BlockSpec((H, H), lambda i: (0, 0)),
            pl.BlockSpec((1, H), lambda i: (0, 0)),
            pl.BlockSpec((H, H), lambda i: (0, 0)),
            pl.BlockSpec((1, H), lambda i: (0, 0)),
            pl.BlockSpec((H, 1), lambda i: (0, 0)),
        ],
        out_specs=[
            pl.BlockSpec((BT, H), lambda i: (i, 0)),
            pl.BlockSpec((BT, X), lambda i: (i, 0)),
        ],
        out_shape=[
            jax.ShapeDtypeStruct((EC, H), jnp.float32),
            jax.ShapeDtypeStruct((EC, X), jnp.float32),
        ],
    )(G, DF, edge_attr, W1a, w1rad, W_e2, b_e2, W_c1, b_c1, W_c2)


def _s2_body(row_hbm, f_hbm, fx_hbm, acc2_hbm, accx2_hbm,
             ib, fb, fxb, ib1, fb1, fxb1, acc, accx, sl0, sl1):
    cid = lax.axis_index("c")
    sid = lax.axis_index("s")
    wbase = (sid * NC + cid) * EW

    zero16 = jnp.zeros((16,), jnp.float32)

    def zrow(r, _):
        for ch in range(H // 16):
            fb[r, pl.ds(ch * 16, 16)] = zero16
        fxb[r, :] = zero16
        return ()

    lax.fori_loop(0, WIN, zrow, ())

    def zchunk(k, _):
        pltpu.sync_copy(fb.at[pl.ds(0, ZR)],
                        acc.at[pl.ds(sid * NPT + k * ZR, ZR)])
        pltpu.sync_copy(fxb.at[pl.ds(0, ZR)],
                        accx.at[pl.ds(sid * NPT + k * ZR, ZR)])
        return ()

    lax.fori_loop(0, NPT // ZR, zchunk, ())
    plsc.subcore_barrier()

    def start_loads(base, ib_, fb_, fxb_, sem):
        return (pltpu.async_copy(row_hbm.at[pl.ds(base, WIN)], ib_, sem),
                pltpu.async_copy(f_hbm.at[pl.ds(base, WIN)], fb_, sem),
                pltpu.async_copy(fx_hbm.at[pl.ds(base, WIN)], fxb_, sem))

    def scatter(ib_, fb_, fxb_):
        pltpu.sync_copy(fb_, acc.at[ib_], add=True)
        pltpu.sync_copy(fxb_, accx.at[ib_], add=True)

    def body(k, _):
        b0 = wbase + 2 * k * WIN
        h0 = start_loads(b0, ib, fb, fxb, sl0)
        h1 = start_loads(b0 + WIN, ib1, fb1, fxb1, sl1)
        for hh in h0:
            hh.wait()
        scatter(ib, fb, fxb)
        for hh in h1:
            hh.wait()
        scatter(ib1, fb1, fxb1)
        return ()

    lax.fori_loop(0, NFULL // 2, body, ())

    lastb = wbase + (NFULL - 1) * WIN
    h0 = start_loads(lastb, ib, fb, fxb, sl0)
    for hh in h0:
        hh.wait()
    scatter(ib, fb, fxb)

    base = wbase + NFULL * WIN
    pltpu.sync_copy(row_hbm.at[pl.ds(base, TAIL)], ib1.at[pl.ds(0, TAIL)])
    ibs = ib1.at[pl.ds(0, TAIL)]
    pltpu.sync_copy(f_hbm.at[pl.ds(base, TAIL)], fb1.at[pl.ds(0, TAIL)])
    pltpu.sync_copy(fx_hbm.at[pl.ds(base, TAIL)], fxb1.at[pl.ds(0, TAIL)])
    pltpu.sync_copy(fb1.at[pl.ds(0, TAIL)], acc.at[ibs], add=True)
    pltpu.sync_copy(fxb1.at[pl.ds(0, TAIL)], accx.at[ibs], add=True)

    plsc.subcore_barrier()

    def flush(k, _):
        off = sid * NPT + k * ZR
        pltpu.sync_copy(acc.at[pl.ds(off, ZR)], acc2_hbm.at[cid, pl.ds(off, ZR)])
        pltpu.sync_copy(accx.at[pl.ds(off, ZR)],
                        accx2_hbm.at[cid, pl.ds(off, ZR)])
        return ()

    lax.fori_loop(0, NPT // ZR, flush, ())


def _s2(row, F, FX):
    mesh = plsc.VectorSubcoreMesh(core_axis_name="c", subcore_axis_name="s")
    return pl.kernel(
        _s2_body,
        out_type=[
            jax.ShapeDtypeStruct((NC, N, H), jnp.float32),
            jax.ShapeDtypeStruct((NC, N, X), jnp.float32),
        ],
        name="s2_scatter",
        mesh=mesh,
        compiler_params=_SC_PARAMS,
        scratch_types=[
            pltpu.VMEM((WIN,), jnp.int32),
            pltpu.VMEM((WIN, H), jnp.float32),
            pltpu.VMEM((WIN, X), jnp.float32),
            pltpu.VMEM((WIN,), jnp.int32),
            pltpu.VMEM((WIN, H), jnp.float32),
            pltpu.VMEM((WIN, X), jnp.float32),
            pltpu.VMEM_SHARED((N, H), jnp.float32),
            pltpu.VMEM_SHARED((N, X), jnp.float32),
            pltpu.SemaphoreType.DMA,
            pltpu.SemaphoreType.DMA,
        ],
    )(row, F, FX)


def _t3_body(*refs):
    acc_refs = refs[:2 * CHUNKS]
    (h_ref, c_ref, wn1h_ref, wn1a_ref, bn1_ref, wn2_ref, bn2_ref,
     ho_ref, co_ref) = refs[2 * CHUNKS:]
    agg_h = sum(r[0] + r[1] for r in acc_refs[0::2])
    accx = sum(r[0] + r[1] for r in acc_refs[1::2])
    sums = accx[:, :3]
    cnt = accx[:, 3:4]
    h = h_ref[...]
    t = jax.nn.silu(h @ wn1h_ref[...] + agg_h @ wn1a_ref[...] + bn1_ref[...])
    ho_ref[...] = h + t @ wn2_ref[...] + bn2_ref[...]
    co_ref[...] = c_ref[...] + sums / jnp.maximum(cnt, 1.0)


def _t3(accs, h, coord, Wn1h, Wn1a, b_n1, W_n2, b_n2):
    BN = 2000
    acc_specs = []
    for _ in range(CHUNKS):
        acc_specs += [
            pl.BlockSpec((NC, BN, H), lambda i: (0, i, 0)),
            pl.BlockSpec((NC, BN, X), lambda i: (0, i, 0)),
        ]
    return pl.pallas_call(
        _t3_body,
        grid=(N // BN,),
        in_specs=acc_specs + [
            pl.BlockSpec((BN, D), lambda i: (i, 0)),
            pl.BlockSpec((BN, 3), lambda i: (i, 0)),
            pl.BlockSpec((D, H), lambda i: (0, 0)),
            pl.BlockSpec((H, H), lambda i: (0, 0)),
            pl.BlockSpec((1, H), lambda i: (0, 0)),
            pl.BlockSpec((H, D), lambda i: (0, 0)),
            pl.BlockSpec((1, D), lambda i: (0, 0)),
        ],
        out_specs=[
            pl.BlockSpec((BN, D), lambda i: (i, 0)),
            pl.BlockSpec((BN, 3), lambda i: (i, 0)),
        ],
        out_shape=[
            jax.ShapeDtypeStruct((N, D), jnp.float32),
            jax.ShapeDtypeStruct((N, 3), jnp.float32),
        ],
    )(*accs, h, coord, Wn1h, Wn1a, b_n1, W_n2, b_n2)


@jax.jit
def kernel(h, edge_index, coord, edge_attr, W_e1, b_e1, W_e2, b_e2,
           W_n1, b_n1, W_n2, b_n2, W_c1, b_c1, W_c2):
    row = edge_index[0]
    col = edge_index[1]

    W1r = W_e1[:D]
    W1c = W_e1[D:2 * D]
    w1rad = W_e1[2 * D:2 * D + 1]
    W1a = W_e1[2 * D + 1:]

    perm = []
    for g in range(H // 32):
        perm += [32 * g + 2 * j for j in range(16)]
        perm += [32 * g + 2 * j + 1 for j in range(16)]
    perm = jnp.array(perm, dtype=jnp.int32)
    w1rad = w1rad[:, perm]
    W1a = W1a[:, perm]
    W_e2p = W_e2[perm, :]

    CX = jnp.pad(coord, ((0, 0), (0, X - 3)))

    P, Q = _t1(h, W1r, W1c, b_e1.reshape(1, H))

    accs = []
    for c in range(CHUNKS):
        sl = slice(c * EC, (c + 1) * EC)
        G, DF = _s1(row[sl], col[sl], P, Q, CX)
        F, FX = _t2(G, DF, edge_attr[sl], W1a, w1rad, W_e2p,
                    b_e2.reshape(1, H), W_c1, b_c1.reshape(1, H), W_c2)
        accs += list(_s2(row[sl], F, FX))

    h_out, coord_out = _t3(accs, h, coord, W_n1[:D], W_n1[D:],
                           b_n1.reshape(1, H), W_n2, b_n2.reshape(1, D))
    return (h_out, coord_out, edge_attr)
